# baseline (speedup 1.0000x reference)
import functools

import jax
import jax.numpy as jnp
from jax import lax
from jax.experimental import pallas as pl
from jax.experimental.pallas import tpu as pltpu
from jax.experimental.pallas import tpu_sc as plsc

B = 16384
S = 200
D = 64
HALF = S // 2
NW = 32
ROWS_PER_W = B // NW
G = 2
GS = G * S
CH = 16
NBUF = 8
NCHUNK = ROWS_PER_W // (G * CH)


def _pool_body(x_flat_hbm, table_flat_hbm, out_hbm, idx_v, gat_v, out_v,
               sem_idx, sem_g0, sem_g1, sem_g2, sem_g3, sem_g4, sem_g5,
               sem_g6, sem_g7, sem_o0, sem_o1):
    x_hbm = x_flat_hbm
    table_hbm = table_flat_hbm
    wid = lax.axis_index("s") * 2 + lax.axis_index("c")
    base = wid * (ROWS_PER_W // G)
    sem_g = (sem_g0, sem_g1, sem_g2, sem_g3, sem_g4, sem_g5, sem_g6,
             sem_g7)
    sem_o = (sem_o0, sem_o1)

    pltpu.async_copy(x_hbm.at[pl.ds(base, CH)], idx_v.at[0], sem_idx)

    def issue_row(c2, r, bslot):
        pltpu.async_copy(table_hbm.at[idx_v.at[c2, r]],
                         gat_v.at[bslot], sem_g[bslot])

    def wait_row(c2, r, bslot):
        pltpu.make_async_copy(table_hbm.at[idx_v.at[c2, r]],
                              gat_v.at[bslot], sem_g[bslot]).wait()

    def reduce_row(c2, r, bslot):
        himask = jnp.full((16,), 0xFFFF0000, jnp.uint32)
        for g in range(G):
            def rbody(s, acc, g=g):
                nxt = list(acc)
                for h in (0, 1):
                    for c in (0, 1):
                        v = gat_v[bslot, g * S + HALF * h + s,
                                  pl.ds(c * 16, 16)]
                        lo = lax.bitcast_convert_type(v << 16,
                                                      jnp.float32)
                        hi = lax.bitcast_convert_type(v & himask,
                                                      jnp.float32)
                        k = (h * 2 + c) * 2
                        nxt[k] = nxt[k] + lo
                        nxt[k + 1] = nxt[k + 1] + hi
                return tuple(nxt)

            acc = lax.fori_loop(
                0, HALF, rbody,
                tuple(jnp.zeros((16,), jnp.float32) for _ in range(8)))
            for c in (0, 1):
                for part in (0, 1):
                    out_v[c2, G * r + g, pl.ds(part * 32 + c * 16, 16)] = (
                        acc[c * 2 + part] + acc[4 + c * 2 + part])

    def process_chunk(q, c2):
        row0 = base + q * CH
        pltpu.make_async_copy(x_hbm.at[pl.ds(row0, CH)], idx_v.at[c2],
                              sem_idx).wait()

        @pl.when(q < NCHUNK - 1)
        def _():
            pltpu.async_copy(x_hbm.at[pl.ds(row0 + CH, CH)],
                             idx_v.at[1 - c2], sem_idx)

        @pl.when(q >= 2)
        def _():
            pltpu.make_async_copy(out_v.at[c2],
                                  out_hbm.at[pl.ds((row0 - 2 * CH) * G,
                                                   G * CH)],
                                  sem_o[c2]).wait()

        for bslot in range(NBUF):
            issue_row(c2, bslot, bslot)

        def jbody(j, carry):
            for bslot in range(NBUF):
                r = NBUF * j + bslot
                wait_row(c2, r, bslot)
                reduce_row(c2, r, bslot)
                issue_row(c2, r + NBUF, bslot)
            return carry

        lax.fori_loop(0, CH // NBUF - 1, jbody, 0)
        for bslot in range(NBUF):
            r = CH - NBUF + bslot
            wait_row(c2, r, bslot)
            reduce_row(c2, r, bslot)

        pltpu.async_copy(out_v.at[c2], out_hbm.at[pl.ds(row0 * G, G * CH)],
                         sem_o[c2])

    def qbody(i, carry):
        process_chunk(2 * i, 0)
        process_chunk(2 * i + 1, 1)
        return carry

    lax.fori_loop(0, NCHUNK // 2, qbody, 0)

    for c2 in (0, 1):
        q = NCHUNK - 2 + c2
        row0 = base + q * CH
        pltpu.make_async_copy(out_v.at[c2],
                              out_hbm.at[pl.ds(row0 * G, G * CH)],
                              sem_o[c2]).wait()


_pool_call = functools.partial(
    pl.kernel,
    out_type=jax.ShapeDtypeStruct((B, D), jnp.float32),
    mesh=plsc.VectorSubcoreMesh(core_axis_name="c", subcore_axis_name="s"),
    compiler_params=pltpu.CompilerParams(use_tc_tiling_on_sc=False),
    scratch_types=[
        pltpu.VMEM((2, CH, GS), jnp.int32),
        pltpu.VMEM((NBUF, GS, 32), jnp.uint32),
        pltpu.VMEM((2, G * CH, D), jnp.float32),
        pltpu.SemaphoreType.DMA,
        pltpu.SemaphoreType.DMA,
        pltpu.SemaphoreType.DMA,
        pltpu.SemaphoreType.DMA,
        pltpu.SemaphoreType.DMA,
        pltpu.SemaphoreType.DMA,
        pltpu.SemaphoreType.DMA,
        pltpu.SemaphoreType.DMA,
        pltpu.SemaphoreType.DMA,
        pltpu.SemaphoreType.DMA,
        pltpu.SemaphoreType.DMA,
    ],
)(_pool_body)


TN = 8192
TLOG = 13


def _tr_body(tin_ref, tout_ref):
    blk = tin_ref[...].astype(jnp.bfloat16)
    stk = jnp.concatenate([blk[:, q * TN:(q + 1) * TN] for q in range(4)],
                          axis=0)
    eye = (lax.broadcasted_iota(jnp.int32, (4 * D, 4 * D), 0)
           == lax.broadcasted_iota(jnp.int32, (4 * D, 4 * D), 1)
           ).astype(jnp.bfloat16)
    y = lax.dot_general(stk, eye, (((0,), (0,)), ((), ())),
                        preferred_element_type=jnp.float32)
    yu = lax.bitcast_convert_type(y, jnp.uint32)
    for q in range(4):
        lo = yu[:, q * D:q * D + 32]
        hi = yu[:, q * D + 32:(q + 1) * D]
        tout_ref[:, q * 32:(q + 1) * 32] = (
            lax.shift_right_logical(lo, jnp.uint32(16))
            | (hi & jnp.uint32(0xFFFF0000)))


def _fc_body(p_ref, wt_ref, b_ref, o_ref):
    o_ref[...] = jnp.dot(jnp.maximum(p_ref[...], 0.0), wt_ref[...],
                         preferred_element_type=jnp.float32) + b_ref[...]


def kernel(x, table, W, b):
    vocab = table.shape[0]
    nblk = (vocab + 4 * TN - 1) // (4 * TN)
    t2 = pl.pallas_call(
        _tr_body,
        grid=(nblk,),
        in_specs=[pl.BlockSpec((D, 4 * TN), lambda i: (0, i))],
        out_specs=pl.BlockSpec((TN, 128), lambda i: (i, 0)),
        out_shape=jax.ShapeDtypeStruct((nblk * TN, 128), jnp.uint32),
    )(table.T)
    t_lin = t2.reshape(nblk * TN * 4, 32)
    xr = (x & ~(4 * TN - 1)) | ((x & (TN - 1)) << 2) | ((x >> TLOG) & 3)
    xr = xr.reshape(B * S // 128, 128)

    pooled = _pool_call(xr.reshape(B // G, GS), t_lin)
    wt = W.T * (1.0 / S)
    b2 = b.reshape(1, -1)
    nc = W.shape[0]
    blk = 1024
    return pl.pallas_call(
        _fc_body,
        grid=(B // blk,),
        in_specs=[pl.BlockSpec((blk, D), lambda i: (i, 0)),
                  pl.BlockSpec((D, nc), lambda i: (0, 0)),
                  pl.BlockSpec((1, nc), lambda i: (0, 0))],
        out_specs=pl.BlockSpec((blk, nc), lambda i: (i, 0)),
        out_shape=jax.ShapeDtypeStruct((B, nc), jnp.float32),
    )(pooled, wt, b2)


# speedup vs baseline: 1.0677x; 1.0677x over previous
import functools

import jax
import jax.numpy as jnp
from jax import lax
from jax.experimental import pallas as pl
from jax.experimental.pallas import tpu as pltpu
from jax.experimental.pallas import tpu_sc as plsc

B = 16384
S = 200
D = 64
HALF = S // 2
NW = 32
ROWS_PER_W = B // NW
G = 2
GS = G * S
CH = 32
NBUF = 4
NCHUNK = ROWS_PER_W // (G * CH)


def _pool_body(x_flat_hbm, table_flat_hbm, out_hbm, idx_v, gat_v, out_v,
               sem_idx, sem_g0, sem_g1, sem_g2, sem_g3, sem_o0, sem_o1):
    x_hbm = x_flat_hbm
    table_hbm = table_flat_hbm
    wid = lax.axis_index("s") * 2 + lax.axis_index("c")
    base = wid * (ROWS_PER_W // G)
    sem_g = (sem_g0, sem_g1, sem_g2, sem_g3)
    sem_o = (sem_o0, sem_o1)

    pltpu.async_copy(x_hbm.at[pl.ds(base, CH)], idx_v.at[0], sem_idx)

    def issue_row(c2, r, bslot):
        pltpu.async_copy(table_hbm.at[idx_v.at[c2, r]],
                         gat_v.at[bslot], sem_g[bslot])

    def wait_row(c2, r, bslot):
        pltpu.make_async_copy(table_hbm.at[idx_v.at[c2, r]],
                              gat_v.at[bslot], sem_g[bslot]).wait()

    def reduce_row(c2, r, bslot):
        himask = jnp.full((16,), 0xFFFF0000, jnp.uint32)
        for g in range(G):
            def rbody(s, acc, g=g):
                nxt = list(acc)
                for h in (0, 1):
                    for c in (0, 1):
                        v = gat_v[bslot, g * S + HALF * h + s,
                                  pl.ds(c * 16, 16)]
                        lo = lax.bitcast_convert_type(v << 16,
                                                      jnp.float32)
                        hi = lax.bitcast_convert_type(v & himask,
                                                      jnp.float32)
                        k = (h * 2 + c) * 2
                        nxt[k] = nxt[k] + lo
                        nxt[k + 1] = nxt[k + 1] + hi
                return tuple(nxt)

            acc = lax.fori_loop(
                0, HALF, rbody,
                tuple(jnp.zeros((16,), jnp.float32) for _ in range(8)))
            for c in (0, 1):
                for part in (0, 1):
                    out_v[c2, G * r + g, pl.ds(part * 32 + c * 16, 16)] = (
                        acc[c * 2 + part] + acc[4 + c * 2 + part])

    def process_chunk(q, c2):
        row0 = base + q * CH
        pltpu.make_async_copy(x_hbm.at[pl.ds(row0, CH)], idx_v.at[c2],
                              sem_idx).wait()

        @pl.when(q < NCHUNK - 1)
        def _():
            pltpu.async_copy(x_hbm.at[pl.ds(row0 + CH, CH)],
                             idx_v.at[1 - c2], sem_idx)

        @pl.when(q >= 2)
        def _():
            pltpu.make_async_copy(out_v.at[c2],
                                  out_hbm.at[pl.ds((row0 - 2 * CH) * G,
                                                   G * CH)],
                                  sem_o[c2]).wait()

        for bslot in range(NBUF):
            issue_row(c2, bslot, bslot)

        def jbody(j, carry):
            for bslot in range(NBUF):
                r = NBUF * j + bslot
                wait_row(c2, r, bslot)
                reduce_row(c2, r, bslot)
                issue_row(c2, r + NBUF, bslot)
            return carry

        lax.fori_loop(0, CH // NBUF - 1, jbody, 0)
        for bslot in range(NBUF):
            r = CH - NBUF + bslot
            wait_row(c2, r, bslot)
            reduce_row(c2, r, bslot)

        pltpu.async_copy(out_v.at[c2], out_hbm.at[pl.ds(row0 * G, G * CH)],
                         sem_o[c2])

    def qbody(i, carry):
        process_chunk(2 * i, 0)
        process_chunk(2 * i + 1, 1)
        return carry

    lax.fori_loop(0, NCHUNK // 2, qbody, 0)

    for c2 in (0, 1):
        q = NCHUNK - 2 + c2
        row0 = base + q * CH
        pltpu.make_async_copy(out_v.at[c2],
                              out_hbm.at[pl.ds(row0 * G, G * CH)],
                              sem_o[c2]).wait()


_pool_call = functools.partial(
    pl.kernel,
    out_type=jax.ShapeDtypeStruct((B, D), jnp.float32),
    mesh=plsc.VectorSubcoreMesh(core_axis_name="c", subcore_axis_name="s"),
    compiler_params=pltpu.CompilerParams(use_tc_tiling_on_sc=False),
    scratch_types=[
        pltpu.VMEM((2, CH, GS), jnp.int32),
        pltpu.VMEM((NBUF, GS, 32), jnp.uint32),
        pltpu.VMEM((2, G * CH, D), jnp.float32),
        pltpu.SemaphoreType.DMA,
        pltpu.SemaphoreType.DMA,
        pltpu.SemaphoreType.DMA,
        pltpu.SemaphoreType.DMA,
        pltpu.SemaphoreType.DMA,
        pltpu.SemaphoreType.DMA,
        pltpu.SemaphoreType.DMA,
    ],
)(_pool_body)


TN = 8192
TLOG = 13


def _tr_body(tin_ref, tout_ref):
    blk = tin_ref[...].astype(jnp.bfloat16)
    stk = jnp.concatenate([blk[:, q * TN:(q + 1) * TN] for q in range(4)],
                          axis=0)
    eye = (lax.broadcasted_iota(jnp.int32, (4 * D, 4 * D), 0)
           == lax.broadcasted_iota(jnp.int32, (4 * D, 4 * D), 1)
           ).astype(jnp.bfloat16)
    y = lax.dot_general(stk, eye, (((0,), (0,)), ((), ())),
                        preferred_element_type=jnp.float32)
    yu = lax.bitcast_convert_type(y, jnp.uint32)
    for q in range(4):
        lo = yu[:, q * D:q * D + 32]
        hi = yu[:, q * D + 32:(q + 1) * D]
        tout_ref[:, q * 32:(q + 1) * 32] = (
            lax.shift_right_logical(lo, jnp.uint32(16))
            | (hi & jnp.uint32(0xFFFF0000)))


def _fc_body(p_ref, wt_ref, b_ref, o_ref):
    o_ref[...] = jnp.dot(jnp.maximum(p_ref[...], 0.0), wt_ref[...],
                         preferred_element_type=jnp.float32) + b_ref[...]


def kernel(x, table, W, b):
    vocab = table.shape[0]
    nblk = (vocab + 4 * TN - 1) // (4 * TN)
    t2 = pl.pallas_call(
        _tr_body,
        grid=(nblk,),
        in_specs=[pl.BlockSpec((D, 4 * TN), lambda i: (0, i))],
        out_specs=pl.BlockSpec((TN, 128), lambda i: (i, 0)),
        out_shape=jax.ShapeDtypeStruct((nblk * TN, 128), jnp.uint32),
    )(table.T)
    t_lin = t2.reshape(nblk * TN * 4, 32)
    xr = (x & ~(4 * TN - 1)) | ((x & (TN - 1)) << 2) | ((x >> TLOG) & 3)
    xr = xr.reshape(B * S // 128, 128)

    pooled = _pool_call(xr.reshape(B // G, GS), t_lin)
    wt = W.T * (1.0 / S)
    b2 = b.reshape(1, -1)
    nc = W.shape[0]
    blk = 1024
    return pl.pallas_call(
        _fc_body,
        grid=(B // blk,),
        in_specs=[pl.BlockSpec((blk, D), lambda i: (i, 0)),
                  pl.BlockSpec((D, nc), lambda i: (0, 0)),
                  pl.BlockSpec((1, nc), lambda i: (0, 0))],
        out_specs=pl.BlockSpec((blk, nc), lambda i: (i, 0)),
        out_shape=jax.ShapeDtypeStruct((B, nc), jnp.float32),
    )(pooled, wt, b2)
